# baseline (speedup 1.0000x reference)
import jax
import jax.numpy as jnp
from jax.experimental import pallas as pl

_N = 10000
_NP = 10240
_G = 256
_L = 40
_F = 64
_K = 16
_S = 3
_R = 80
_GEO = -1.0 / (2.0 * 0.1 ** 2)
_TEX = -5.0
_BIGI = 2 ** 30


def _normalize_kernel(f_ref, c_ref):
    f = f_ref[...]
    nrm = jnp.sqrt(jnp.sum(f * f, axis=1, keepdims=True))
    z = f / jnp.maximum(nrm, 1e-12)
    c_ref[0:_N, 0:_F] = z
    c_ref[0:_N, _F:2 * _F] = z * z
    c_ref[_N:_NP, :] = jnp.zeros((_NP - _N, 2 * _F), jnp.float32)


def _loss_kernel(aux_row_ref, aux3_ref, c_full_ref, c_row_ref,
                 out_ref):
    i = pl.program_id(0)
    inf = jnp.float32(jnp.inf)

    px = aux_row_ref[:, 0:1]
    py = aux_row_ref[:, 1:2]
    pz = aux_row_ref[:, 2:3]
    rowg = i * _R + jax.lax.broadcasted_iota(jnp.int32, (_R, 1), 0)

    dx = px.reshape(_R, 1, 1) - aux3_ref[0:1, :, :]
    dy = py.reshape(_R, 1, 1) - aux3_ref[1:2, :, :]
    dz = pz.reshape(_R, 1, 1) - aux3_ref[2:3, :, :]
    d2g = dx * dx + dy * dy + dz * dz
    g_iota = jax.lax.broadcasted_iota(jnp.int32, (1, _L, _G), 2)
    m_iota = jax.lax.broadcasted_iota(jnp.int32, (1, _L, _G), 1)
    c3 = m_iota * _G + g_iota
    dist3 = jnp.where(c3 == rowg.reshape(_R, 1, 1), inf, d2g)

    cand_v = []
    cand_c = []
    mems = []
    for _ in range(_S):
        if mems:
            excl = m_iota == mems[0]
            for prev in mems[1:]:
                excl = excl | (m_iota == prev)
            dcur = jnp.where(excl, inf, dist3)
        else:
            dcur = dist3
        m = jnp.min(dcur, axis=1, keepdims=True)
        mem = jnp.min(jnp.where(dcur == m, m_iota, _BIGI), axis=1,
                      keepdims=True)
        mems.append(mem)
        cand_v.append(m)
        cand_c.append(mem * _G + g_iota[:, 0:1, :])
    vs = jnp.concatenate(cand_v, axis=1)
    cs = jnp.concatenate(cand_c, axis=1)

    def merge_body(_, carry):
        vw, _, _ = carry
        mv = jnp.min(jnp.min(vw, axis=1, keepdims=True), axis=2,
                     keepdims=True)
        jc = jnp.min(jnp.min(jnp.where(vw == mv, cs, _BIGI), axis=1,
                             keepdims=True), axis=2, keepdims=True)
        vw = jnp.where(cs == jc, inf, vw)
        return vw, mv, jc

    init = (vs, jnp.zeros((_R, 1, 1), jnp.float32),
            jnp.zeros((_R, 1, 1), jnp.int32))
    _, t_fast, j_fast = jax.lax.fori_loop(0, _K, merge_body, init)

    lexle = (vs < t_fast) | ((vs == t_fast) & (cs <= j_fast))
    cnt = jnp.sum(lexle.astype(jnp.int32), axis=1, keepdims=True)
    cmax = jnp.max(jnp.max(cnt, axis=2, keepdims=True), axis=0,
                   keepdims=True)
    bad = cmax[0, 0, 0] >= _S

    def _slow():
        sx = px.reshape(_R, 1, 1) - aux3_ref[0:1, :, :]
        sy = py.reshape(_R, 1, 1) - aux3_ref[1:2, :, :]
        sz = pz.reshape(_R, 1, 1) - aux3_ref[2:3, :, :]
        ds = sx * sx + sy * sy + sz * sz
        ds = jnp.where(c3 == rowg.reshape(_R, 1, 1), inf, ds)

        def body(_, carry):
            dws, _, _ = carry
            mv = jnp.min(jnp.min(dws, axis=1, keepdims=True), axis=2,
                         keepdims=True)
            jc = jnp.min(jnp.min(jnp.where(dws == mv, c3, _BIGI), axis=1,
                                 keepdims=True), axis=2, keepdims=True)
            dws = jnp.where(c3 == jc, inf, dws)
            return dws, mv, jc

        init_s = (ds, jnp.zeros((_R, 1, 1), jnp.float32),
                  jnp.zeros((_R, 1, 1), jnp.int32))
        _, t2, j2 = jax.lax.fori_loop(0, _K, body, init_s)
        return t2, j2

    thr3, jst3 = jax.lax.cond(bad, _slow, lambda: (t_fast, j_fast))

    sel3 = (dist3 < thr3) | ((dist3 == thr3) & (c3 <= jst3))
    same3 = aux_row_ref[:, 6:7].reshape(_R, 1, 1) == aux3_ref[6:7, :, :]
    rx = aux_row_ref[:, 3:4].reshape(_R, 1, 1) - aux3_ref[3:4, :, :]
    ry = aux_row_ref[:, 4:5].reshape(_R, 1, 1) - aux3_ref[4:5, :, :]
    rz = aux_row_ref[:, 5:6].reshape(_R, 1, 1) - aux3_ref[5:6, :, :]
    drgb3 = rx * rx + ry * ry + rz * rz
    w3 = jnp.exp(d2g * _GEO + drgb3 * _TEX)
    wm3 = jnp.where(sel3 & same3, w3, 0.0)

    roww = jnp.sum(jnp.sum(wm3, axis=1, keepdims=True), axis=2,
                   keepdims=True).reshape(_R, 1)
    wmat = wm3.reshape(_R, _NP)
    a = jnp.dot(wmat, c_full_ref[...],
                preferred_element_type=jnp.float32)
    a1 = a[:, 0:_F]
    b = jnp.sum(a[:, _F:2 * _F], axis=1, keepdims=True)
    zr = c_row_ref[:, 0:_F]
    qr = jnp.sum(c_row_ref[:, _F:2 * _F], axis=1, keepdims=True)
    li = qr * roww + b - 2.0 * jnp.sum(zr * a1, axis=1, keepdims=True)
    part = jnp.sum(li, axis=0, keepdims=True)

    @pl.when(i == 0)
    def _():
        out_ref[...] = jnp.zeros((1, 1), jnp.float32)

    out_ref[...] += part


def kernel(features, pos, rgb, target):
    n = pos.shape[0]
    c = pl.pallas_call(
        _normalize_kernel,
        out_shape=jax.ShapeDtypeStruct((_NP, 2 * _F), jnp.float32),
    )(features)

    aux_row = jnp.concatenate(
        [pos, rgb, target.astype(jnp.float32)[:, None],
         jnp.zeros((n, 1), jnp.float32)], axis=1)
    aux_row = jnp.concatenate(
        [aux_row, jnp.full((_NP - n, 8), 1e6, jnp.float32)], axis=0)
    aux2 = aux_row.T
    aux3 = aux2.reshape(8, _L, _G)

    grid = (n + _R - 1) // _R
    total = pl.pallas_call(
        _loss_kernel,
        grid=(grid,),
        in_specs=[
            pl.BlockSpec((_R, 8), lambda i: (i, 0)),
            pl.BlockSpec((8, _L, _G), lambda i: (0, 0, 0)),
            pl.BlockSpec((_NP, 2 * _F), lambda i: (0, 0)),
            pl.BlockSpec((_R, 2 * _F), lambda i: (i, 0)),
        ],
        out_specs=pl.BlockSpec((1, 1), lambda i: (0, 0)),
        out_shape=jax.ShapeDtypeStruct((1, 1), jnp.float32),
    )(aux_row, aux3, c, c)

    return total[0, 0] / jnp.float32(n)


# speedup vs baseline: 1.8733x; 1.8733x over previous
import jax
import jax.numpy as jnp
from jax.experimental import pallas as pl

_N = 10000
_NP = 10240
_G = 128
_L = 80
_F = 64
_K = 16
_S = 4
_R = 80
_GEO = -1.0 / (2.0 * 0.1 ** 2)
_TEX = -5.0
_BIGI = 2 ** 30


def _normalize_kernel(f_ref, c_ref):
    f = f_ref[...]
    nrm = jnp.sqrt(jnp.sum(f * f, axis=1, keepdims=True))
    z = f / jnp.maximum(nrm, 1e-12)
    c_ref[0:_N, 0:_F] = z
    c_ref[0:_N, _F:2 * _F] = z * z
    c_ref[_N:_NP, :] = jnp.zeros((_NP - _N, 2 * _F), jnp.float32)


def _loss_kernel(aux_row_ref, aux3_ref, c_full_ref, c_row_ref,
                 out_ref):
    i = pl.program_id(0)
    inf = jnp.float32(jnp.inf)

    px = aux_row_ref[:, 0:1]
    py = aux_row_ref[:, 1:2]
    pz = aux_row_ref[:, 2:3]
    rowg = i * _R + jax.lax.broadcasted_iota(jnp.int32, (_R, 1), 0)

    dx = px.reshape(_R, 1, 1) - aux3_ref[0:1, :, :]
    dy = py.reshape(_R, 1, 1) - aux3_ref[1:2, :, :]
    dz = pz.reshape(_R, 1, 1) - aux3_ref[2:3, :, :]
    d2g = dx * dx + dy * dy + dz * dz
    g_iota = jax.lax.broadcasted_iota(jnp.int32, (1, _L, _G), 2)
    m_iota = jax.lax.broadcasted_iota(jnp.int32, (1, _L, _G), 1)
    c3 = m_iota * _G + g_iota
    dist3 = jnp.where(c3 == rowg.reshape(_R, 1, 1), inf, d2g)

    cand_v = []
    cand_c = []
    mems = []
    for _ in range(_S):
        if mems:
            excl = m_iota == mems[0]
            for prev in mems[1:]:
                excl = excl | (m_iota == prev)
            dcur = jnp.where(excl, inf, dist3)
        else:
            dcur = dist3
        m = jnp.min(dcur, axis=1, keepdims=True)
        mem = jnp.min(jnp.where(dcur == m, m_iota, _BIGI), axis=1,
                      keepdims=True)
        mems.append(mem)
        cand_v.append(m)
        cand_c.append(mem * _G + g_iota[:, 0:1, :])
    vs = jnp.concatenate(cand_v, axis=1)
    cs = jnp.concatenate(cand_c, axis=1)

    def merge_body(_, carry):
        vw, _, _ = carry
        mv = jnp.min(jnp.min(vw, axis=1, keepdims=True), axis=2,
                     keepdims=True)
        jc = jnp.min(jnp.min(jnp.where(vw == mv, cs, _BIGI), axis=1,
                             keepdims=True), axis=2, keepdims=True)
        vw = jnp.where(cs == jc, inf, vw)
        return vw, mv, jc

    init = (vs, jnp.zeros((_R, 1, 1), jnp.float32),
            jnp.zeros((_R, 1, 1), jnp.int32))
    _, t_fast, j_fast = jax.lax.fori_loop(0, _K, merge_body, init)

    lexle = (vs < t_fast) | ((vs == t_fast) & (cs <= j_fast))
    cnt = jnp.sum(lexle.astype(jnp.int32), axis=1, keepdims=True)
    cmax = jnp.max(jnp.max(cnt, axis=2, keepdims=True), axis=0,
                   keepdims=True)
    bad = cmax[0, 0, 0] >= _S

    def _slow():
        sx = px.reshape(_R, 1, 1) - aux3_ref[0:1, :, :]
        sy = py.reshape(_R, 1, 1) - aux3_ref[1:2, :, :]
        sz = pz.reshape(_R, 1, 1) - aux3_ref[2:3, :, :]
        ds = sx * sx + sy * sy + sz * sz
        ds = jnp.where(c3 == rowg.reshape(_R, 1, 1), inf, ds)

        def body(_, carry):
            dws, _, _ = carry
            mv = jnp.min(jnp.min(dws, axis=1, keepdims=True), axis=2,
                         keepdims=True)
            jc = jnp.min(jnp.min(jnp.where(dws == mv, c3, _BIGI), axis=1,
                                 keepdims=True), axis=2, keepdims=True)
            dws = jnp.where(c3 == jc, inf, dws)
            return dws, mv, jc

        init_s = (ds, jnp.zeros((_R, 1, 1), jnp.float32),
                  jnp.zeros((_R, 1, 1), jnp.int32))
        _, t2, j2 = jax.lax.fori_loop(0, _K, body, init_s)
        return t2, j2

    thr3, jst3 = jax.lax.cond(bad, _slow, lambda: (t_fast, j_fast))

    sel3 = (dist3 < thr3) | ((dist3 == thr3) & (c3 <= jst3))
    same3 = aux_row_ref[:, 6:7].reshape(_R, 1, 1) == aux3_ref[6:7, :, :]
    rx = aux_row_ref[:, 3:4].reshape(_R, 1, 1) - aux3_ref[3:4, :, :]
    ry = aux_row_ref[:, 4:5].reshape(_R, 1, 1) - aux3_ref[4:5, :, :]
    rz = aux_row_ref[:, 5:6].reshape(_R, 1, 1) - aux3_ref[5:6, :, :]
    drgb3 = rx * rx + ry * ry + rz * rz
    w3 = jnp.exp(d2g * _GEO + drgb3 * _TEX)
    wm3 = jnp.where(sel3 & same3, w3, 0.0)

    roww = jnp.sum(jnp.sum(wm3, axis=1, keepdims=True), axis=2,
                   keepdims=True).reshape(_R, 1)
    wmat = wm3.reshape(_R, _NP)
    a = jnp.dot(wmat, c_full_ref[...],
                preferred_element_type=jnp.float32)
    a1 = a[:, 0:_F]
    b = jnp.sum(a[:, _F:2 * _F], axis=1, keepdims=True)
    zr = c_row_ref[:, 0:_F]
    qr = jnp.sum(c_row_ref[:, _F:2 * _F], axis=1, keepdims=True)
    li = qr * roww + b - 2.0 * jnp.sum(zr * a1, axis=1, keepdims=True)
    part = jnp.sum(li, axis=0, keepdims=True)

    @pl.when(i == 0)
    def _():
        out_ref[...] = jnp.zeros((1, 1), jnp.float32)

    out_ref[...] += part


def kernel(features, pos, rgb, target):
    n = pos.shape[0]
    c = pl.pallas_call(
        _normalize_kernel,
        out_shape=jax.ShapeDtypeStruct((_NP, 2 * _F), jnp.float32),
    )(features)

    aux_row = jnp.concatenate(
        [pos, rgb, target.astype(jnp.float32)[:, None],
         jnp.zeros((n, 1), jnp.float32)], axis=1)
    aux_row = jnp.concatenate(
        [aux_row, jnp.full((_NP - n, 8), 1e6, jnp.float32)], axis=0)
    aux2 = aux_row.T
    aux3 = aux2.reshape(8, _L, _G)

    grid = (n + _R - 1) // _R
    total = pl.pallas_call(
        _loss_kernel,
        grid=(grid,),
        in_specs=[
            pl.BlockSpec((_R, 8), lambda i: (i, 0)),
            pl.BlockSpec((8, _L, _G), lambda i: (0, 0, 0)),
            pl.BlockSpec((_NP, 2 * _F), lambda i: (0, 0)),
            pl.BlockSpec((_R, 2 * _F), lambda i: (i, 0)),
        ],
        out_specs=pl.BlockSpec((1, 1), lambda i: (0, 0)),
        out_shape=jax.ShapeDtypeStruct((1, 1), jnp.float32),
    )(aux_row, aux3, c, c)

    return total[0, 0] / jnp.float32(n)


# speedup vs baseline: 1.9085x; 1.0188x over previous
import jax
import jax.numpy as jnp
from jax.experimental import pallas as pl

_N = 10000
_NP = 10240
_G = 128
_L = 80
_F = 64
_K = 16
_S = 4
_R = 96
_GEO = -1.0 / (2.0 * 0.1 ** 2)
_TEX = -5.0
_BIGI = 2 ** 30


def _normalize_kernel(f_ref, c_ref):
    f = f_ref[...]
    nrm = jnp.sqrt(jnp.sum(f * f, axis=1, keepdims=True))
    z = f / jnp.maximum(nrm, 1e-12)
    c_ref[0:_N, 0:_F] = z
    c_ref[0:_N, _F:2 * _F] = z * z
    c_ref[_N:_NP, :] = jnp.zeros((_NP - _N, 2 * _F), jnp.float32)


def _loss_kernel(aux_row_ref, aux3_ref, c_full_ref, c_row_ref,
                 out_ref):
    i = pl.program_id(0)
    inf = jnp.float32(jnp.inf)

    px = aux_row_ref[:, 0:1]
    py = aux_row_ref[:, 1:2]
    pz = aux_row_ref[:, 2:3]
    rowg = i * _R + jax.lax.broadcasted_iota(jnp.int32, (_R, 1), 0)

    dx = px.reshape(_R, 1, 1) - aux3_ref[0:1, :, :]
    dy = py.reshape(_R, 1, 1) - aux3_ref[1:2, :, :]
    dz = pz.reshape(_R, 1, 1) - aux3_ref[2:3, :, :]
    d2g = dx * dx + dy * dy + dz * dz
    g_iota = jax.lax.broadcasted_iota(jnp.int32, (1, _L, _G), 2)
    m_iota = jax.lax.broadcasted_iota(jnp.int32, (1, _L, _G), 1)
    c3 = m_iota * _G + g_iota
    dist3 = jnp.where(c3 == rowg.reshape(_R, 1, 1), inf, d2g)

    dw = dist3
    cand_v = []
    cand_c = []
    for _ in range(_S):
        m = jnp.min(dw, axis=1, keepdims=True)
        mem = jnp.min(jnp.where(dw == m, m_iota, _BIGI), axis=1,
                      keepdims=True)
        dw = jnp.where(m_iota == mem, inf, dw)
        cand_v.append(m)
        cand_c.append(mem * _G + g_iota[:, 0:1, :])
    vs = jnp.concatenate(cand_v, axis=1)
    cs = jnp.concatenate(cand_c, axis=1)

    def merge_body(_, carry):
        vw, _, _ = carry
        mv = jnp.min(jnp.min(vw, axis=1, keepdims=True), axis=2,
                     keepdims=True)
        jc = jnp.min(jnp.min(jnp.where(vw == mv, cs, _BIGI), axis=1,
                             keepdims=True), axis=2, keepdims=True)
        vw = jnp.where(cs == jc, inf, vw)
        return vw, mv, jc

    init = (vs, jnp.zeros((_R, 1, 1), jnp.float32),
            jnp.zeros((_R, 1, 1), jnp.int32))
    _, t_fast, j_fast = jax.lax.fori_loop(0, _K, merge_body, init)

    lexle = (vs < t_fast) | ((vs == t_fast) & (cs <= j_fast))
    cnt = jnp.sum(lexle.astype(jnp.int32), axis=1, keepdims=True)
    cmax = jnp.max(jnp.max(cnt, axis=2, keepdims=True), axis=0,
                   keepdims=True)
    bad = cmax[0, 0, 0] >= _S

    def _slow():
        sx = px.reshape(_R, 1, 1) - aux3_ref[0:1, :, :]
        sy = py.reshape(_R, 1, 1) - aux3_ref[1:2, :, :]
        sz = pz.reshape(_R, 1, 1) - aux3_ref[2:3, :, :]
        ds = sx * sx + sy * sy + sz * sz
        ds = jnp.where(c3 == rowg.reshape(_R, 1, 1), inf, ds)

        def body(_, carry):
            dws, _, _ = carry
            mv = jnp.min(jnp.min(dws, axis=1, keepdims=True), axis=2,
                         keepdims=True)
            jc = jnp.min(jnp.min(jnp.where(dws == mv, c3, _BIGI), axis=1,
                                 keepdims=True), axis=2, keepdims=True)
            dws = jnp.where(c3 == jc, inf, dws)
            return dws, mv, jc

        init_s = (ds, jnp.zeros((_R, 1, 1), jnp.float32),
                  jnp.zeros((_R, 1, 1), jnp.int32))
        _, t2, j2 = jax.lax.fori_loop(0, _K, body, init_s)
        return t2, j2

    thr3, jst3 = jax.lax.cond(bad, _slow, lambda: (t_fast, j_fast))

    sel3 = (dist3 < thr3) | ((dist3 == thr3) & (c3 <= jst3))
    same3 = aux_row_ref[:, 6:7].reshape(_R, 1, 1) == aux3_ref[6:7, :, :]
    rx = aux_row_ref[:, 3:4].reshape(_R, 1, 1) - aux3_ref[3:4, :, :]
    ry = aux_row_ref[:, 4:5].reshape(_R, 1, 1) - aux3_ref[4:5, :, :]
    rz = aux_row_ref[:, 5:6].reshape(_R, 1, 1) - aux3_ref[5:6, :, :]
    drgb3 = rx * rx + ry * ry + rz * rz
    w3 = jnp.exp(d2g * _GEO + drgb3 * _TEX)
    wm3 = jnp.where(sel3 & same3, w3, 0.0)

    roww = jnp.sum(jnp.sum(wm3, axis=1, keepdims=True), axis=2,
                   keepdims=True).reshape(_R, 1)
    wmat = wm3.reshape(_R, _NP)
    a = jnp.dot(wmat, c_full_ref[...],
                preferred_element_type=jnp.float32)
    a1 = a[:, 0:_F]
    b = jnp.sum(a[:, _F:2 * _F], axis=1, keepdims=True)
    zr = c_row_ref[:, 0:_F]
    qr = jnp.sum(c_row_ref[:, _F:2 * _F], axis=1, keepdims=True)
    li = qr * roww + b - 2.0 * jnp.sum(zr * a1, axis=1, keepdims=True)
    part = jnp.sum(li, axis=0, keepdims=True)

    @pl.when(i == 0)
    def _():
        out_ref[...] = jnp.zeros((1, 1), jnp.float32)

    out_ref[...] += part


def kernel(features, pos, rgb, target):
    n = pos.shape[0]
    c = pl.pallas_call(
        _normalize_kernel,
        out_shape=jax.ShapeDtypeStruct((_NP, 2 * _F), jnp.float32),
    )(features)

    aux_row = jnp.concatenate(
        [pos, rgb, target.astype(jnp.float32)[:, None],
         jnp.zeros((n, 1), jnp.float32)], axis=1)
    aux_row = jnp.concatenate(
        [aux_row, jnp.full((_NP - n, 8), 1e6, jnp.float32)], axis=0)
    aux2 = aux_row.T
    aux3 = aux2.reshape(8, _L, _G)

    grid = (n + _R - 1) // _R
    total = pl.pallas_call(
        _loss_kernel,
        grid=(grid,),
        in_specs=[
            pl.BlockSpec((_R, 8), lambda i: (i, 0)),
            pl.BlockSpec((8, _L, _G), lambda i: (0, 0, 0)),
            pl.BlockSpec((_NP, 2 * _F), lambda i: (0, 0)),
            pl.BlockSpec((_R, 2 * _F), lambda i: (i, 0)),
        ],
        out_specs=pl.BlockSpec((1, 1), lambda i: (0, 0)),
        out_shape=jax.ShapeDtypeStruct((1, 1), jnp.float32),
    )(aux_row, aux3, c, c)

    return total[0, 0] / jnp.float32(n)
